# baseline (speedup 1.0000x reference)
import math

import jax
import jax.numpy as jnp
from jax.experimental import pallas as pl
from jax.experimental.pallas import tpu as pltpu

_POS = 1
_NEG = 0
_ALPHA = 0.5
_GAMMA = 1.5
_BB = math.e ** (_GAMMA / _ALPHA) - 1.0

_CE_CHUNK = 4000
_BOX_CHUNK = 4000


def _ce_body(pc_ref, ce0_ref, ce1_ref):
    x = pc_ref[...].reshape(_CE_CHUNK // 125, 125, pc_ref.shape[1])
    mx = jnp.max(x, axis=2)
    lse = mx + jnp.log(jnp.sum(jnp.exp(x - mx[:, :, None]), axis=2))
    ce0_ref[...] = lse - x[:, :, _NEG]
    ce1_ref[...] = lse - x[:, :, _POS]


def _box_body(pb_ref, gb_ref, rs_ref):
    d = jnp.abs(pb_ref[0] - gb_ref[0]).reshape(_BOX_CHUNK // 125, 125, 4)
    bl = jnp.where(
        d < 1.0,
        _ALPHA / _BB * (_BB * d + 1.0) * jnp.log(_BB * d + 1.0) - _ALPHA * d,
        _GAMMA * d + _GAMMA / _BB - _ALPHA,
    )
    rs_ref[...] = jnp.sum(bl, axis=2)


def _mask_body(gtc_ref, ce0_ref, ce1_ref, rs_ref, negce_ref, stats_ref):
    g = gtc_ref[...]
    posm = g == _POS
    negm = g == _NEG

    negce_ref[...] = jnp.where(
        negm, jnp.maximum(ce0_ref[...], 0.0), 0.0)

    posf = posm.astype(jnp.float32)
    rs = rs_ref[...]
    stats_ref[0] = jnp.sum(posf)
    stats_ref[1] = jnp.sum(negm.astype(jnp.float32))
    stats_ref[2] = jnp.sum(jnp.where(posm, ce1_ref[...], 0.0))
    stats_ref[3] = jnp.sum(posf * rs)


def _select_body(stats_ref, neg_ref, out_ref):
    pos_cnt = stats_ref[0]
    neg_cnt = stats_ref[1]
    cls_pos = stats_ref[2]
    loc_sum = stats_ref[3]

    kf = jnp.minimum(pos_cnt, neg_cnt)
    k = kf.astype(jnp.int32)

    x = neg_ref[...]
    u = jax.lax.bitcast_convert_type(x, jnp.int32)

    def body(j, prefix):
        trial = prefix | (jnp.int32(1) << (jnp.int32(30) - j))
        cnt = jnp.sum((u >= trial).astype(jnp.int32))
        return jnp.where(cnt >= k, trial, prefix)

    t_bits = jax.lax.fori_loop(0, 31, body, jnp.int32(0))

    gt = u > t_bits
    cnt_gt = jnp.sum(gt.astype(jnp.int32))
    sum_gt = jnp.sum(jnp.where(gt, x, 0.0))
    t_val = jax.lax.bitcast_convert_type(t_bits, jnp.float32)
    cls_neg = jnp.where(
        k > 0, sum_gt + (kf - cnt_gt.astype(jnp.float32)) * t_val, 0.0
    )

    has_pos = pos_cnt > 0.0
    ns = pos_cnt + kf
    out_ref[0] = jnp.where(has_pos, loc_sum / jnp.maximum(pos_cnt, 1.0), 0.0)
    out_ref[1] = jnp.where(
        has_pos, (cls_pos + cls_neg) / jnp.maximum(ns, 1.0), 0.0
    )


def kernel(predicted_boxes, predicted_classes, gt_bboxes, gt_classes):
    b, a, c = predicted_classes.shape
    n = b * a
    rows = n // 125
    ce_steps = n // _CE_CHUNK
    ce_rows = _CE_CHUNK // 125
    per_b = a // _BOX_CHUNK
    box_steps = b * per_b

    pc = predicted_classes.reshape(n, c)
    gtc_flat = gt_classes.reshape(rows, 125)

    ce0, ce1 = pl.pallas_call(
        _ce_body,
        grid=(ce_steps,),
        in_specs=[pl.BlockSpec((_CE_CHUNK, c), lambda t: (t, 0))],
        out_specs=[
            pl.BlockSpec((ce_rows, 125), lambda t: (t, 0)),
            pl.BlockSpec((ce_rows, 125), lambda t: (t, 0)),
        ],
        out_shape=[
            jax.ShapeDtypeStruct((rows, 125), jnp.float32),
            jax.ShapeDtypeStruct((rows, 125), jnp.float32),
        ],
        compiler_params=pltpu.CompilerParams(
            dimension_semantics=("arbitrary",),
        ),
    )(pc)

    rs = pl.pallas_call(
        _box_body,
        grid=(box_steps,),
        in_specs=[
            pl.BlockSpec((1, _BOX_CHUNK, 4),
                         lambda t: (t // per_b, t % per_b, 0)),
            pl.BlockSpec((1, _BOX_CHUNK, 4),
                         lambda t: (t // per_b, t % per_b, 0)),
        ],
        out_specs=pl.BlockSpec((_BOX_CHUNK // 125, 125), lambda t: (t, 0)),
        out_shape=jax.ShapeDtypeStruct((rows, 125), jnp.float32),
        compiler_params=pltpu.CompilerParams(
            dimension_semantics=("arbitrary",),
        ),
    )(predicted_boxes, gt_bboxes)

    negce, stats = pl.pallas_call(
        _mask_body,
        in_specs=[
            pl.BlockSpec((rows, 125), lambda: (0, 0)),
            pl.BlockSpec((rows, 125), lambda: (0, 0)),
            pl.BlockSpec((rows, 125), lambda: (0, 0)),
            pl.BlockSpec((rows, 125), lambda: (0, 0)),
        ],
        out_specs=[
            pl.BlockSpec((rows, 125), lambda: (0, 0)),
            pl.BlockSpec(memory_space=pltpu.SMEM),
        ],
        out_shape=[
            jax.ShapeDtypeStruct((rows, 125), jnp.float32),
            jax.ShapeDtypeStruct((4,), jnp.float32),
        ],
    )(gtc_flat, ce0, ce1, rs)

    out = pl.pallas_call(
        _select_body,
        in_specs=[
            pl.BlockSpec(memory_space=pltpu.SMEM),
            pl.BlockSpec((rows, 125), lambda: (0, 0)),
        ],
        out_specs=pl.BlockSpec(memory_space=pltpu.SMEM),
        out_shape=jax.ShapeDtypeStruct((2,), jnp.float32),
    )(stats, negce)

    return (out[0], out[1])


# speedup vs baseline: 2.1491x; 2.1491x over previous
import math

import jax
import jax.numpy as jnp
from jax.experimental import pallas as pl
from jax.experimental.pallas import tpu as pltpu

_POS = 1
_NEG = 0
_ALPHA = 0.5
_GAMMA = 1.5
_BB = math.e ** (_GAMMA / _ALPHA) - 1.0

_CE_CHUNK = 4000
_BOX_CHUNK = 4000


def _ce_body(pc_ref, ce0_ref, ce1_ref):
    x = pc_ref[...]
    mx = jnp.max(x, axis=1)
    lse = mx + jnp.log(jnp.sum(jnp.exp(x - mx[:, None]), axis=1))
    ce0_ref[...] = (lse - x[:, _NEG]).reshape(_CE_CHUNK // 125, 125)
    ce1_ref[...] = (lse - x[:, _POS]).reshape(_CE_CHUNK // 125, 125)


def _box_body(pb_ref, gb_ref, rs_ref):
    d = jnp.abs(pb_ref[0] - gb_ref[0])
    bl = jnp.where(
        d < 1.0,
        _ALPHA / _BB * (_BB * d + 1.0) * jnp.log(_BB * d + 1.0) - _ALPHA * d,
        _GAMMA * d + _GAMMA / _BB - _ALPHA,
    )
    rs_ref[...] = jnp.sum(bl, axis=1).reshape(_BOX_CHUNK // 125, 125)


def _mask_body(gtc_ref, ce0_ref, ce1_ref, rs_ref, negce_ref, stats_ref):
    g = gtc_ref[...]
    posm = g == jnp.float32(_POS)
    negm = g == jnp.float32(_NEG)

    negce_ref[...] = jnp.where(
        negm, jnp.maximum(ce0_ref[...], 0.0), 0.0)

    posf = posm.astype(jnp.float32)
    rs = rs_ref[...]
    stats_ref[0] = jnp.sum(posf)
    stats_ref[1] = jnp.sum(negm.astype(jnp.float32))
    stats_ref[2] = jnp.sum(jnp.where(posm, ce1_ref[...], 0.0))
    stats_ref[3] = jnp.sum(posf * rs)


def _select_body(stats_ref, neg_ref, out_ref):
    pos_cnt = stats_ref[0]
    neg_cnt = stats_ref[1]
    cls_pos = stats_ref[2]
    loc_sum = stats_ref[3]

    kf = jnp.minimum(pos_cnt, neg_cnt)
    k = kf.astype(jnp.int32)

    x = neg_ref[...]
    u = jax.lax.bitcast_convert_type(x, jnp.int32)

    def body(j, prefix):
        trial = prefix | (jnp.int32(1) << (jnp.int32(30) - j))
        cnt = jnp.sum((u >= trial).astype(jnp.int32))
        return jnp.where(cnt >= k, trial, prefix)

    t_bits = jax.lax.fori_loop(0, 31, body, jnp.int32(0))

    gt = u > t_bits
    cnt_gt = jnp.sum(gt.astype(jnp.int32))
    sum_gt = jnp.sum(jnp.where(gt, x, 0.0))
    t_val = jax.lax.bitcast_convert_type(t_bits, jnp.float32)
    cls_neg = jnp.where(
        k > 0, sum_gt + (kf - cnt_gt.astype(jnp.float32)) * t_val, 0.0
    )

    has_pos = pos_cnt > 0.0
    ns = pos_cnt + kf
    out_ref[0] = jnp.where(has_pos, loc_sum / jnp.maximum(pos_cnt, 1.0), 0.0)
    out_ref[1] = jnp.where(
        has_pos, (cls_pos + cls_neg) / jnp.maximum(ns, 1.0), 0.0
    )


def kernel(predicted_boxes, predicted_classes, gt_bboxes, gt_classes):
    b, a, c = predicted_classes.shape
    n = b * a
    rows = n // 125
    ce_steps = n // _CE_CHUNK
    ce_rows = _CE_CHUNK // 125
    per_b = a // _BOX_CHUNK
    box_steps = b * per_b

    pc = predicted_classes.reshape(n, c)
    gtc_flat = gt_classes.astype(jnp.float32).reshape(rows, 125)

    ce0, ce1 = pl.pallas_call(
        _ce_body,
        grid=(ce_steps,),
        in_specs=[pl.BlockSpec((_CE_CHUNK, c), lambda t: (t, 0))],
        out_specs=[
            pl.BlockSpec((ce_rows, 125), lambda t: (t, 0)),
            pl.BlockSpec((ce_rows, 125), lambda t: (t, 0)),
        ],
        out_shape=[
            jax.ShapeDtypeStruct((rows, 125), jnp.float32),
            jax.ShapeDtypeStruct((rows, 125), jnp.float32),
        ],
        compiler_params=pltpu.CompilerParams(
            dimension_semantics=("arbitrary",),
        ),
    )(pc)

    rs = pl.pallas_call(
        _box_body,
        grid=(box_steps,),
        in_specs=[
            pl.BlockSpec((1, _BOX_CHUNK, 4),
                         lambda t: (t // per_b, t % per_b, 0)),
            pl.BlockSpec((1, _BOX_CHUNK, 4),
                         lambda t: (t // per_b, t % per_b, 0)),
        ],
        out_specs=pl.BlockSpec((_BOX_CHUNK // 125, 125), lambda t: (t, 0)),
        out_shape=jax.ShapeDtypeStruct((rows, 125), jnp.float32),
        compiler_params=pltpu.CompilerParams(
            dimension_semantics=("arbitrary",),
        ),
    )(predicted_boxes, gt_bboxes)

    negce, stats = pl.pallas_call(
        _mask_body,
        in_specs=[
            pl.BlockSpec((rows, 125), lambda: (0, 0)),
            pl.BlockSpec((rows, 125), lambda: (0, 0)),
            pl.BlockSpec((rows, 125), lambda: (0, 0)),
            pl.BlockSpec((rows, 125), lambda: (0, 0)),
        ],
        out_specs=[
            pl.BlockSpec((rows, 125), lambda: (0, 0)),
            pl.BlockSpec(memory_space=pltpu.SMEM),
        ],
        out_shape=[
            jax.ShapeDtypeStruct((rows, 125), jnp.float32),
            jax.ShapeDtypeStruct((4,), jnp.float32),
        ],
    )(gtc_flat, ce0, ce1, rs)

    out = pl.pallas_call(
        _select_body,
        in_specs=[
            pl.BlockSpec(memory_space=pltpu.SMEM),
            pl.BlockSpec((rows, 125), lambda: (0, 0)),
        ],
        out_specs=pl.BlockSpec(memory_space=pltpu.SMEM),
        out_shape=jax.ShapeDtypeStruct((2,), jnp.float32),
    )(stats, negce)

    return (out[0], out[1])


# speedup vs baseline: 2.1718x; 1.0106x over previous
import math

import jax
import jax.numpy as jnp
from jax.experimental import pallas as pl
from jax.experimental.pallas import tpu as pltpu

_POS = 1
_NEG = 0
_ALPHA = 0.5
_GAMMA = 1.5
_BB = math.e ** (_GAMMA / _ALPHA) - 1.0

_CE_CHUNK = 8000
_BOX_CHUNK = 4000


def _ce_body(pc_ref, ce0_ref, ce1_ref):
    x = pc_ref[...]
    mx = jnp.max(x, axis=1)
    lse = mx + jnp.log(jnp.sum(jnp.exp(x - mx[:, None]), axis=1))
    ce0_ref[...] = (lse - x[:, _NEG]).reshape(_CE_CHUNK // 125, 125)
    ce1_ref[...] = (lse - x[:, _POS]).reshape(_CE_CHUNK // 125, 125)


def _box_body(pb_ref, gb_ref, rs_ref):
    d = jnp.abs(pb_ref[0] - gb_ref[0])
    bl = jnp.where(
        d < 1.0,
        _ALPHA / _BB * (_BB * d + 1.0) * jnp.log(_BB * d + 1.0) - _ALPHA * d,
        _GAMMA * d + _GAMMA / _BB - _ALPHA,
    )
    rs_ref[...] = jnp.sum(bl, axis=1).reshape(_BOX_CHUNK // 125, 125)


def _mask_body(gtc_ref, ce0_ref, ce1_ref, rs_ref, negce_ref, stats_ref):
    g = gtc_ref[...]
    posm = g == jnp.float32(_POS)
    negm = g == jnp.float32(_NEG)

    negce_ref[...] = jnp.where(
        negm, jnp.maximum(ce0_ref[...], 0.0), 0.0)

    posf = posm.astype(jnp.float32)
    stats_ref[0] = jnp.sum(posf)
    stats_ref[1] = jnp.sum(negm.astype(jnp.float32))
    stats_ref[2] = jnp.sum(jnp.where(posm, ce1_ref[...], 0.0))
    stats_ref[3] = jnp.sum(posf * rs_ref[...])


def _select_body(stats_ref, neg_ref, out_ref):
    pos_cnt = stats_ref[0]
    neg_cnt = stats_ref[1]
    cls_pos = stats_ref[2]
    loc_sum = stats_ref[3]

    kf = jnp.minimum(pos_cnt, neg_cnt)
    k = kf.astype(jnp.int32)

    x = neg_ref[...]
    u = jax.lax.bitcast_convert_type(x, jnp.int32)

    def body(j, prefix):
        trial = prefix | (jnp.int32(1) << (jnp.int32(30) - j))
        cnt = jnp.sum((u >= trial).astype(jnp.int32))
        return jnp.where(cnt >= k, trial, prefix)

    t_bits = jax.lax.fori_loop(0, 31, body, jnp.int32(0))

    gt = u > t_bits
    cnt_gt = jnp.sum(gt.astype(jnp.int32))
    sum_gt = jnp.sum(jnp.where(gt, x, 0.0))
    t_val = jax.lax.bitcast_convert_type(t_bits, jnp.float32)
    cls_neg = jnp.where(
        k > 0, sum_gt + (kf - cnt_gt.astype(jnp.float32)) * t_val, 0.0
    )

    has_pos = pos_cnt > 0.0
    ns = pos_cnt + kf
    out_ref[0] = jnp.where(has_pos, loc_sum / jnp.maximum(pos_cnt, 1.0), 0.0)
    out_ref[1] = jnp.where(
        has_pos, (cls_pos + cls_neg) / jnp.maximum(ns, 1.0), 0.0
    )


def kernel(predicted_boxes, predicted_classes, gt_bboxes, gt_classes):
    b, a, c = predicted_classes.shape
    n = b * a
    rows = n // 125
    ce_steps = n // _CE_CHUNK
    ce_rows = _CE_CHUNK // 125
    per_b = a // _BOX_CHUNK
    box_steps = b * per_b

    pc = predicted_classes.reshape(n, c)
    gtc_flat = gt_classes.astype(jnp.float32).reshape(rows, 125)

    ce0, ce1 = pl.pallas_call(
        _ce_body,
        grid=(ce_steps,),
        in_specs=[pl.BlockSpec((_CE_CHUNK, c), lambda t: (t, 0))],
        out_specs=[
            pl.BlockSpec((ce_rows, 125), lambda t: (t, 0)),
            pl.BlockSpec((ce_rows, 125), lambda t: (t, 0)),
        ],
        out_shape=[
            jax.ShapeDtypeStruct((rows, 125), jnp.float32),
            jax.ShapeDtypeStruct((rows, 125), jnp.float32),
        ],
        compiler_params=pltpu.CompilerParams(
            dimension_semantics=("arbitrary",),
        ),
    )(pc)

    rs = pl.pallas_call(
        _box_body,
        grid=(box_steps,),
        in_specs=[
            pl.BlockSpec((1, _BOX_CHUNK, 4),
                         lambda t: (t // per_b, t % per_b, 0)),
            pl.BlockSpec((1, _BOX_CHUNK, 4),
                         lambda t: (t // per_b, t % per_b, 0)),
        ],
        out_specs=pl.BlockSpec((_BOX_CHUNK // 125, 125), lambda t: (t, 0)),
        out_shape=jax.ShapeDtypeStruct((rows, 125), jnp.float32),
        compiler_params=pltpu.CompilerParams(
            dimension_semantics=("arbitrary",),
        ),
    )(predicted_boxes, gt_bboxes)

    negce, stats = pl.pallas_call(
        _mask_body,
        in_specs=[
            pl.BlockSpec((rows, 125), lambda: (0, 0)),
            pl.BlockSpec((rows, 125), lambda: (0, 0)),
            pl.BlockSpec((rows, 125), lambda: (0, 0)),
            pl.BlockSpec((rows, 125), lambda: (0, 0)),
        ],
        out_specs=[
            pl.BlockSpec((rows, 125), lambda: (0, 0)),
            pl.BlockSpec(memory_space=pltpu.SMEM),
        ],
        out_shape=[
            jax.ShapeDtypeStruct((rows, 125), jnp.float32),
            jax.ShapeDtypeStruct((4,), jnp.float32),
        ],
    )(gtc_flat, ce0, ce1, rs)

    out = pl.pallas_call(
        _select_body,
        in_specs=[
            pl.BlockSpec(memory_space=pltpu.SMEM),
            pl.BlockSpec((rows, 125), lambda: (0, 0)),
        ],
        out_specs=pl.BlockSpec(memory_space=pltpu.SMEM),
        out_shape=jax.ShapeDtypeStruct((2,), jnp.float32),
    )(stats, negce)

    return (out[0], out[1])


# speedup vs baseline: 2.1722x; 1.0002x over previous
import math

import jax
import jax.numpy as jnp
from jax.experimental import pallas as pl
from jax.experimental.pallas import tpu as pltpu

_POS = 1
_NEG = 0
_ALPHA = 0.5
_GAMMA = 1.5
_BB = math.e ** (_GAMMA / _ALPHA) - 1.0

_CE_CHUNK = 8000
_BOX_CHUNK = 4000


def _ce_body(pc_ref, ce0_ref, ce1_ref):
    x = pc_ref[...]
    mx = jnp.max(x, axis=1)
    lse = mx + jnp.log(jnp.sum(jnp.exp(x - mx[:, None]), axis=1))
    ce0_ref[...] = (lse - x[:, _NEG]).reshape(_CE_CHUNK // 125, 125)
    ce1_ref[...] = (lse - x[:, _POS]).reshape(_CE_CHUNK // 125, 125)


def _box_body(pb_ref, gb_ref, rs_ref):
    d = jnp.abs(pb_ref[0] - gb_ref[0])
    bl = jnp.where(
        d < 1.0,
        _ALPHA / _BB * (_BB * d + 1.0) * jnp.log(_BB * d + 1.0) - _ALPHA * d,
        _GAMMA * d + _GAMMA / _BB - _ALPHA,
    )
    rs_ref[...] = jnp.sum(bl, axis=1).reshape(_BOX_CHUNK // 125, 125)


def _mask_body(gtc_ref, ce0_ref, ce1_ref, rs_ref, negce_ref, stats_ref):
    g = gtc_ref[...]
    posm = g == jnp.int8(_POS)
    negm = g == jnp.int8(_NEG)

    negce_ref[...] = jnp.where(
        negm, jnp.maximum(ce0_ref[...], 0.0), 0.0)

    posf = posm.astype(jnp.float32)
    stats_ref[0] = jnp.sum(posf)
    stats_ref[1] = jnp.sum(negm.astype(jnp.float32))
    stats_ref[2] = jnp.sum(jnp.where(posm, ce1_ref[...], 0.0))
    stats_ref[3] = jnp.sum(posf * rs_ref[...])


def _select_body(stats_ref, neg_ref, out_ref):
    pos_cnt = stats_ref[0]
    neg_cnt = stats_ref[1]
    cls_pos = stats_ref[2]
    loc_sum = stats_ref[3]

    kf = jnp.minimum(pos_cnt, neg_cnt)
    k = kf.astype(jnp.int32)

    x = neg_ref[...]
    u = jax.lax.bitcast_convert_type(x, jnp.int32)

    def body(j, prefix):
        trial = prefix | (jnp.int32(1) << (jnp.int32(30) - j))
        cnt = jnp.sum((u >= trial).astype(jnp.int32))
        return jnp.where(cnt >= k, trial, prefix)

    t_bits = jax.lax.fori_loop(0, 31, body, jnp.int32(0))

    gt = u > t_bits
    cnt_gt = jnp.sum(gt.astype(jnp.int32))
    sum_gt = jnp.sum(jnp.where(gt, x, 0.0))
    t_val = jax.lax.bitcast_convert_type(t_bits, jnp.float32)
    cls_neg = jnp.where(
        k > 0, sum_gt + (kf - cnt_gt.astype(jnp.float32)) * t_val, 0.0
    )

    has_pos = pos_cnt > 0.0
    ns = pos_cnt + kf
    out_ref[0] = jnp.where(has_pos, loc_sum / jnp.maximum(pos_cnt, 1.0), 0.0)
    out_ref[1] = jnp.where(
        has_pos, (cls_pos + cls_neg) / jnp.maximum(ns, 1.0), 0.0
    )


def kernel(predicted_boxes, predicted_classes, gt_bboxes, gt_classes):
    b, a, c = predicted_classes.shape
    n = b * a
    rows = n // 125
    ce_steps = n // _CE_CHUNK
    ce_rows = _CE_CHUNK // 125
    per_b = a // _BOX_CHUNK
    box_steps = b * per_b

    pc = predicted_classes.reshape(n, c)
    gtc_flat = gt_classes.astype(jnp.int8).reshape(rows, 125)

    ce0, ce1 = pl.pallas_call(
        _ce_body,
        grid=(ce_steps,),
        in_specs=[pl.BlockSpec((_CE_CHUNK, c), lambda t: (t, 0))],
        out_specs=[
            pl.BlockSpec((ce_rows, 125), lambda t: (t, 0)),
            pl.BlockSpec((ce_rows, 125), lambda t: (t, 0)),
        ],
        out_shape=[
            jax.ShapeDtypeStruct((rows, 125), jnp.float32),
            jax.ShapeDtypeStruct((rows, 125), jnp.float32),
        ],
        compiler_params=pltpu.CompilerParams(
            dimension_semantics=("arbitrary",),
        ),
    )(pc)

    rs = pl.pallas_call(
        _box_body,
        grid=(box_steps,),
        in_specs=[
            pl.BlockSpec((1, _BOX_CHUNK, 4),
                         lambda t: (t // per_b, t % per_b, 0)),
            pl.BlockSpec((1, _BOX_CHUNK, 4),
                         lambda t: (t // per_b, t % per_b, 0)),
        ],
        out_specs=pl.BlockSpec((_BOX_CHUNK // 125, 125), lambda t: (t, 0)),
        out_shape=jax.ShapeDtypeStruct((rows, 125), jnp.float32),
        compiler_params=pltpu.CompilerParams(
            dimension_semantics=("arbitrary",),
        ),
    )(predicted_boxes, gt_bboxes)

    negce, stats = pl.pallas_call(
        _mask_body,
        in_specs=[
            pl.BlockSpec((rows, 125), lambda: (0, 0)),
            pl.BlockSpec((rows, 125), lambda: (0, 0)),
            pl.BlockSpec((rows, 125), lambda: (0, 0)),
            pl.BlockSpec((rows, 125), lambda: (0, 0)),
        ],
        out_specs=[
            pl.BlockSpec((rows, 125), lambda: (0, 0)),
            pl.BlockSpec(memory_space=pltpu.SMEM),
        ],
        out_shape=[
            jax.ShapeDtypeStruct((rows, 125), jnp.float32),
            jax.ShapeDtypeStruct((4,), jnp.float32),
        ],
    )(gtc_flat, ce0, ce1, rs)

    out = pl.pallas_call(
        _select_body,
        in_specs=[
            pl.BlockSpec(memory_space=pltpu.SMEM),
            pl.BlockSpec((rows, 125), lambda: (0, 0)),
        ],
        out_specs=pl.BlockSpec(memory_space=pltpu.SMEM),
        out_shape=jax.ShapeDtypeStruct((2,), jnp.float32),
    )(stats, negce)

    return (out[0], out[1])
